# baseline (speedup 1.0000x reference)
import dataclasses
import functools

import jax
import jax.numpy as jnp
from jax import lax
from jax.experimental import pallas as pl
from jax.experimental.pallas import tpu as pltpu
from jax.experimental.pallas import tpu_sc as plsc

N_NODES = 100000
Q = 16
DT = 0.1

NC = 2
NS = 16
NW = NC * NS
SUB = 128
B = 512
KSUB = B // SUB

N1 = 100352
RPS = N1 // NS

E_PAD = 3211264
E_PER_W = E_PAD // NW
NCHUNK = E_PER_W // B


def _sc_all_body(f_hbm, src_hbm, dst_hbm, w_hbm,
                 acc_out, sd_out,
                 srcv, dstv, wv, rows0, rows1,
                 acc_sh, sem_g, sem_sc):
    c = lax.axis_index("c")
    s = lax.axis_index("s")
    wid = s * NC + c

    iota = lax.iota(jnp.int32, 16)
    pat = jnp.where(iota == 1, 1.0, 0.0).astype(jnp.float32)
    zeros16_i = jnp.zeros((16,), jnp.int32)
    zeros16_f = jnp.zeros((16,), jnp.float32)
    r0 = pl.multiple_of(s * RPS, 8)
    ebase = wid * E_PER_W
    rbase = ebase // SUB

    def zero_slice():
        @pl.loop(0, B)
        def _zrow(j):
            rows0[j] = zeros16_f

        for t in range((RPS + B - 1) // B):
            nr = min(B, RPS - t * B)
            pltpu.sync_copy(rows0.at[pl.ds(0, nr)],
                            acc_sh.at[pl.ds(r0 + t * B, nr)])

    def drain_slice(out):
        pltpu.sync_copy(acc_sh.at[pl.ds(r0, RPS)], out.at[c, pl.ds(r0, RPS)])

    def fire_gathers(rows, half):
        return [
            pltpu.async_copy(f_hbm.at[srcv.at[half * KSUB + j]],
                             rows.at[pl.ds(j * SUB, SUB)], sem_g)
            for j in range(KSUB)
        ]

    def fire_scatters(rows, half):
        return [
            pltpu.async_copy(rows.at[pl.ds(j * SUB, SUB)],
                             acc_sh.at[dstv.at[half * KSUB + j]], sem_sc,
                             add=True)
            for j in range(KSUB)
        ]

    def multiply(rows, half):
        off = half * B

        @plsc.parallel_loop(0, B, unroll=8)
        def _mul(j):
            wb = plsc.load_gather(wv, [zeros16_i + (off + j)])
            rows[j] = rows[j] * wb

    zero_slice()
    plsc.subcore_barrier()

    @pl.loop(0, NCHUNK, step=2)
    def _pipe(i):
        rb = pl.multiple_of(rbase + i * KSUB, 8)
        eb = pl.multiple_of(ebase + i * B, 8)
        pltpu.sync_copy(src_hbm.at[pl.ds(rb, 2 * KSUB)], srcv)
        pltpu.sync_copy(dst_hbm.at[pl.ds(rb, 2 * KSUB)], dstv)
        pltpu.sync_copy(w_hbm.at[pl.ds(eb, 2 * B)], wv)
        g0 = fire_gathers(rows0, 0)
        g1 = fire_gathers(rows1, 1)
        for h in g0:
            h.wait()
        multiply(rows0, 0)
        s0 = fire_scatters(rows0, 0)
        for h in g1:
            h.wait()
        multiply(rows1, 1)
        for h in s0:
            h.wait()
        s1 = fire_scatters(rows1, 1)
        for h in s1:
            h.wait()

    plsc.subcore_barrier()
    drain_slice(acc_out)

    zero_slice()

    @pl.loop(0, B)
    def _pval(j):
        rows0[j] = pat
        rows1[j] = pat
    plsc.subcore_barrier()

    def build(vals, half):
        @plsc.parallel_loop(0, B // 16, unroll=4)
        def _mkval(g):
            w16 = wv[pl.ds(half * B + g * 16, 16)]
            plsc.store_scatter(vals, [g * 16 + iota, zeros16_i], w16)

    @pl.loop(0, NCHUNK, step=2)
    def _chunk(ci):
        rb = pl.multiple_of(rbase + ci * KSUB, 8)
        eb = pl.multiple_of(ebase + ci * B, 8)
        pltpu.sync_copy(dst_hbm.at[pl.ds(rb, 2 * KSUB)], dstv)
        pltpu.sync_copy(w_hbm.at[pl.ds(eb, 2 * B)], wv)

        build(rows0, 0)
        s0 = fire_scatters(rows0, 0)
        build(rows1, 1)
        for h in s0:
            h.wait()
        s1 = fire_scatters(rows1, 1)
        for h in s1:
            h.wait()

    plsc.subcore_barrier()
    drain_slice(sd_out)


_SC_CP = pltpu.CompilerParams(needs_layout_passes=False,
                              use_tc_tiling_on_sc=False)


@jax.jit
def _sc_segment_sums(f, src2, dst2, w1):
    mesh = plsc.VectorSubcoreMesh(core_axis_name="c", subcore_axis_name="s")
    k = pl.kernel(
        _sc_all_body,
        compiler_params=_SC_CP,
        out_type=[jax.ShapeDtypeStruct((NC, N1, Q), jnp.float32),
                  jax.ShapeDtypeStruct((NC, N1, Q), jnp.float32)],
        mesh=mesh,
        scratch_types=[
            pltpu.VMEM((2 * KSUB, SUB), jnp.int32),
            pltpu.VMEM((2 * KSUB, SUB), jnp.int32),
            pltpu.VMEM((2 * B,), jnp.float32),
            pltpu.VMEM((B, Q), jnp.float32),
            pltpu.VMEM((B, Q), jnp.float32),
            pltpu.VMEM_SHARED((N1, Q), jnp.float32),
            pltpu.SemaphoreType.DMA,
            pltpu.SemaphoreType.DMA,
        ],
    )
    return k(f, src2, dst2, w1)


def _combine_body(f_ref, coll_ref, srcterm_ref, acc_ref, sd_ref, xi_ref,
                  out_ref):
    f = f_ref[...]
    a = acc_ref[0] + acc_ref[1]
    sv = sd_ref[0, :, 0:1] + sd_ref[1, :, 0:1]
    deg = sd_ref[0, :, 1:2] + sd_ref[1, :, 1:2]
    deg = jnp.maximum(deg, 1.0)
    xi = xi_ref[...]
    transport = xi * (a - f * sv) / deg
    out_ref[...] = f - DT * (transport - coll_ref[...] + srcterm_ref[...])


@jax.jit
def _tc_combine(f, coll, srcterm, acc, sd, xi):
    R = 1000
    grid = (N_NODES // R,)
    return pl.pallas_call(
        _combine_body,
        grid=grid,
        in_specs=[
            pl.BlockSpec((R, Q), lambda i: (i, 0)),
            pl.BlockSpec((R, Q), lambda i: (i, 0)),
            pl.BlockSpec((R, Q), lambda i: (i, 0)),
            pl.BlockSpec((NC, R, Q), lambda i: (0, i, 0)),
            pl.BlockSpec((NC, R, Q), lambda i: (0, i, 0)),
            pl.BlockSpec((1, Q), lambda i: (0, 0)),
        ],
        out_specs=pl.BlockSpec((R, Q), lambda i: (i, 0)),
        out_shape=jax.ShapeDtypeStruct((N_NODES, Q), jnp.float32),
    )(f, coll, srcterm, acc, sd, xi)


def kernel(f_distribution, collision_term, source_term, edge_index,
           edge_weight, xi_velocities):
    E = edge_weight.shape[0]
    pad = E_PAD - E
    src = jnp.concatenate([edge_index[0], jnp.zeros((pad,), jnp.int32)])
    dst = jnp.concatenate([edge_index[1],
                           jnp.full((pad,), N_NODES, jnp.int32)])
    w = jnp.concatenate([edge_weight, jnp.zeros((pad,), jnp.float32)])
    src2 = src.reshape(E_PAD // SUB, SUB)
    dst2 = dst.reshape(E_PAD // SUB, SUB)
    acc, sd = _sc_segment_sums(f_distribution, src2, dst2, w)
    return _tc_combine(f_distribution, collision_term, source_term, acc, sd,
                       xi_velocities.reshape(1, Q))


# speedup vs baseline: 1.0765x; 1.0765x over previous
import dataclasses
import functools

import jax
import jax.numpy as jnp
from jax import lax
from jax.experimental import pallas as pl
from jax.experimental.pallas import tpu as pltpu
from jax.experimental.pallas import tpu_sc as plsc

N_NODES = 100000
Q = 16
DT = 0.1

NC = 2
NS = 16
NW = NC * NS
SUB = 128
B = 512
KSUB = B // SUB

N1 = 100352
RPS = N1 // NS

E_PAD = 3211264
E_PER_W = E_PAD // NW
NCHUNK = E_PER_W // B


def _sc_all_body(f_hbm, src_hbm, dst_hbm, w_hbm,
                 acc_out, sd_out,
                 srcv, dstv, wv, rows0, rows1,
                 acc_sh, sem_g, sem_sc):
    c = lax.axis_index("c")
    s = lax.axis_index("s")
    wid = s * NC + c

    iota = lax.iota(jnp.int32, 16)
    pat = jnp.where(iota == 1, 1.0, 0.0).astype(jnp.float32)
    zeros16_i = jnp.zeros((16,), jnp.int32)
    zeros16_f = jnp.zeros((16,), jnp.float32)
    r0 = pl.multiple_of(s * RPS, 8)
    ebase = wid * E_PER_W
    rbase = ebase // SUB

    def zero_slice():
        @pl.loop(0, B)
        def _zrow(j):
            rows0[j] = zeros16_f

        for t in range((RPS + B - 1) // B):
            nr = min(B, RPS - t * B)
            pltpu.sync_copy(rows0.at[pl.ds(0, nr)],
                            acc_sh.at[pl.ds(r0 + t * B, nr)])

    def drain_slice(out):
        pltpu.sync_copy(acc_sh.at[pl.ds(r0, RPS)], out.at[c, pl.ds(r0, RPS)])

    def fire_gathers(rows, half):
        return [
            pltpu.async_copy(f_hbm.at[srcv.at[half * KSUB + j]],
                             rows.at[pl.ds(j * SUB, SUB)], sem_g)
            for j in range(KSUB)
        ]

    def fire_scatters(rows, half):
        return [
            pltpu.async_copy(rows.at[pl.ds(j * SUB, SUB)],
                             acc_sh.at[dstv.at[half * KSUB + j]], sem_sc,
                             add=True)
            for j in range(KSUB)
        ]

    def multiply(rows, half):
        off = half * B

        @plsc.parallel_loop(0, B, unroll=8)
        def _mul(j):
            wb = plsc.load_gather(wv, [zeros16_i + (off + j)])
            rows[j] = rows[j] * wb

    zero_slice()
    plsc.subcore_barrier()

    @pl.loop(0, NCHUNK, step=4)
    def _pipe(i):
        rb = pl.multiple_of(rbase + i * KSUB, 8)
        eb = pl.multiple_of(ebase + i * B, 8)
        pltpu.sync_copy(src_hbm.at[pl.ds(rb, 4 * KSUB)], srcv)
        pltpu.sync_copy(dst_hbm.at[pl.ds(rb, 4 * KSUB)], dstv)
        pltpu.sync_copy(w_hbm.at[pl.ds(eb, 4 * B)], wv)
        g0 = fire_gathers(rows0, 0)
        g1 = fire_gathers(rows1, 1)
        for h in g0:
            h.wait()
        multiply(rows0, 0)
        s0 = fire_scatters(rows0, 0)
        for h in g1:
            h.wait()
        multiply(rows1, 1)
        for h in s0:
            h.wait()
        g2 = fire_gathers(rows0, 2)
        s1 = fire_scatters(rows1, 1)
        for h in g2:
            h.wait()
        multiply(rows0, 2)
        for h in s1:
            h.wait()
        g3 = fire_gathers(rows1, 3)
        s2 = fire_scatters(rows0, 2)
        for h in g3:
            h.wait()
        multiply(rows1, 3)
        for h in s2:
            h.wait()
        s3 = fire_scatters(rows1, 3)
        for h in s3:
            h.wait()

    plsc.subcore_barrier()
    drain_slice(acc_out)

    zero_slice()

    @pl.loop(0, B)
    def _pval(j):
        rows0[j] = pat
        rows1[j] = pat
    plsc.subcore_barrier()

    def build(vals, half):
        @plsc.parallel_loop(0, B // 16, unroll=4)
        def _mkval(g):
            w16 = wv[pl.ds(half * B + g * 16, 16)]
            plsc.store_scatter(vals, [g * 16 + iota, zeros16_i], w16)

    @pl.loop(0, NCHUNK, step=4)
    def _chunk(ci):
        rb = pl.multiple_of(rbase + ci * KSUB, 8)
        eb = pl.multiple_of(ebase + ci * B, 8)
        pltpu.sync_copy(dst_hbm.at[pl.ds(rb, 4 * KSUB)], dstv)
        pltpu.sync_copy(w_hbm.at[pl.ds(eb, 4 * B)], wv)

        build(rows0, 0)
        s0 = fire_scatters(rows0, 0)
        build(rows1, 1)
        for h in s0:
            h.wait()
        s1 = fire_scatters(rows1, 1)
        build(rows0, 2)
        for h in s1:
            h.wait()
        s2 = fire_scatters(rows0, 2)
        build(rows1, 3)
        for h in s2:
            h.wait()
        s3 = fire_scatters(rows1, 3)
        for h in s3:
            h.wait()

    plsc.subcore_barrier()
    drain_slice(sd_out)


_SC_CP = pltpu.CompilerParams(needs_layout_passes=False,
                              use_tc_tiling_on_sc=False)


@jax.jit
def _sc_segment_sums(f, src2, dst2, w1):
    mesh = plsc.VectorSubcoreMesh(core_axis_name="c", subcore_axis_name="s")
    k = pl.kernel(
        _sc_all_body,
        compiler_params=_SC_CP,
        out_type=[jax.ShapeDtypeStruct((NC, N1, Q), jnp.float32),
                  jax.ShapeDtypeStruct((NC, N1, Q), jnp.float32)],
        mesh=mesh,
        scratch_types=[
            pltpu.VMEM((4 * KSUB, SUB), jnp.int32),
            pltpu.VMEM((4 * KSUB, SUB), jnp.int32),
            pltpu.VMEM((4 * B,), jnp.float32),
            pltpu.VMEM((B, Q), jnp.float32),
            pltpu.VMEM((B, Q), jnp.float32),
            pltpu.VMEM_SHARED((N1, Q), jnp.float32),
            pltpu.SemaphoreType.DMA,
            pltpu.SemaphoreType.DMA,
        ],
    )
    return k(f, src2, dst2, w1)


def _combine_body(f_ref, coll_ref, srcterm_ref, acc_ref, sd_ref, xi_ref,
                  out_ref):
    f = f_ref[...]
    a = acc_ref[0] + acc_ref[1]
    sv = sd_ref[0, :, 0:1] + sd_ref[1, :, 0:1]
    deg = sd_ref[0, :, 1:2] + sd_ref[1, :, 1:2]
    deg = jnp.maximum(deg, 1.0)
    xi = xi_ref[...]
    transport = xi * (a - f * sv) / deg
    out_ref[...] = f - DT * (transport - coll_ref[...] + srcterm_ref[...])


@jax.jit
def _tc_combine(f, coll, srcterm, acc, sd, xi):
    R = 1000
    grid = (N_NODES // R,)
    return pl.pallas_call(
        _combine_body,
        grid=grid,
        in_specs=[
            pl.BlockSpec((R, Q), lambda i: (i, 0)),
            pl.BlockSpec((R, Q), lambda i: (i, 0)),
            pl.BlockSpec((R, Q), lambda i: (i, 0)),
            pl.BlockSpec((NC, R, Q), lambda i: (0, i, 0)),
            pl.BlockSpec((NC, R, Q), lambda i: (0, i, 0)),
            pl.BlockSpec((1, Q), lambda i: (0, 0)),
        ],
        out_specs=pl.BlockSpec((R, Q), lambda i: (i, 0)),
        out_shape=jax.ShapeDtypeStruct((N_NODES, Q), jnp.float32),
    )(f, coll, srcterm, acc, sd, xi)


def kernel(f_distribution, collision_term, source_term, edge_index,
           edge_weight, xi_velocities):
    E = edge_weight.shape[0]
    pad = E_PAD - E
    src = jnp.concatenate([edge_index[0], jnp.zeros((pad,), jnp.int32)])
    dst = jnp.concatenate([edge_index[1],
                           jnp.full((pad,), N_NODES, jnp.int32)])
    w = jnp.concatenate([edge_weight, jnp.zeros((pad,), jnp.float32)])
    src2 = src.reshape(E_PAD // SUB, SUB)
    dst2 = dst.reshape(E_PAD // SUB, SUB)
    acc, sd = _sc_segment_sums(f_distribution, src2, dst2, w)
    return _tc_combine(f_distribution, collision_term, source_term, acc, sd,
                       xi_velocities.reshape(1, Q))
